# baseline (speedup 1.0000x reference)
import functools

import jax
import jax.numpy as jnp
from jax import lax
from jax.experimental import pallas as pl
from jax.experimental.pallas import tpu as pltpu
from jax.experimental.pallas import tpu_sc as plsc

_TABLE_PAD = 1024


@functools.lru_cache(maxsize=None)
def _sc_gather(B: int, num_steps: int):
    info = plsc.get_sparse_core_info()
    nc, ns, L = info.num_cores, info.num_subcores, info.num_lanes
    nw = nc * ns
    b_per_w = B // nw
    mesh = plsc.VectorSubcoreMesh(core_axis_name="c", subcore_axis_name="s")

    @functools.partial(
        pl.kernel,
        mesh=mesh,
        out_type=(
            jax.ShapeDtypeStruct((B,), jnp.float32),
            jax.ShapeDtypeStruct((B,), jnp.float32),
        ),
        scratch_types=[
            pltpu.VMEM((_TABLE_PAD,), jnp.float32),
            pltpu.VMEM((_TABLE_PAD,), jnp.float32),
            pltpu.VMEM((b_per_w,), jnp.int32),
            pltpu.VMEM((b_per_w,), jnp.float32),
            pltpu.VMEM((b_per_w,), jnp.float32),
        ],
        compiler_params=pltpu.CompilerParams(needs_layout_passes=False),
    )
    def gather(t_hbm, sab_hbm, somab_hbm, sa_hbm, sb_hbm,
               sab_v, somab_v, idx_v, sa_v, sb_v):
        wid = lax.axis_index("s") * nc + lax.axis_index("c")
        base = wid * b_per_w
        pltpu.sync_copy(sab_hbm, sab_v)
        pltpu.sync_copy(somab_hbm, somab_v)
        pltpu.sync_copy(t_hbm.at[pl.ds(base, b_per_w)], idx_v)
        for j in range(b_per_w // L):
            idx = idx_v[pl.ds(j * L, L)]
            idx = jnp.minimum(jnp.maximum(idx, 0), num_steps - 1)
            sa_v[pl.ds(j * L, L)] = plsc.load_gather(sab_v, [idx])
            sb_v[pl.ds(j * L, L)] = plsc.load_gather(somab_v, [idx])
        pltpu.sync_copy(sa_v, sa_hbm.at[pl.ds(base, b_per_w)])
        pltpu.sync_copy(sb_v, sb_hbm.at[pl.ds(base, b_per_w)])

    return gather


def _tc_combine_body(sa_ref, sb_ref, x_ref, n_ref, o_ref):
    o_ref[...] = sa_ref[...] * x_ref[...] + sb_ref[...] * n_ref[...]


@functools.lru_cache(maxsize=None)
def _tc_combine(B: int, C: int, H: int, W: int, R: int):
    data = pl.BlockSpec((R, C, H, W), lambda i: (i, 0, 0, 0))
    scale = pl.BlockSpec((R, 1, 1, 1), lambda i: (i, 0, 0, 0))
    return pl.pallas_call(
        _tc_combine_body,
        grid=(B // R,),
        in_specs=[scale, scale, data, data],
        out_specs=data,
        out_shape=jax.ShapeDtypeStruct((B, C, H, W), jnp.float32),
    )


def kernel(x_start, noise, t, sqrt_alpha_bars, sqrt_one_minus_alpha_bars):
    B, C, H, W = x_start.shape
    num_steps = sqrt_alpha_bars.shape[0]
    pad = _TABLE_PAD - num_steps
    sab = jnp.pad(sqrt_alpha_bars, (0, pad))
    somab = jnp.pad(sqrt_one_minus_alpha_bars, (0, pad))
    sa, sb = _sc_gather(B, num_steps)(t, sab, somab)
    x_t = _tc_combine(B, C, H, W, 8)(
        sa.reshape(B, 1, 1, 1), sb.reshape(B, 1, 1, 1), x_start, noise)
    return (x_t, noise)


# speedup vs baseline: 1.3958x; 1.3958x over previous
import functools

import jax
import jax.numpy as jnp
from jax import lax
from jax.experimental import pallas as pl
from jax.experimental.pallas import tpu as pltpu
from jax.experimental.pallas import tpu_sc as plsc

_TABLE_PAD = 1024


@functools.lru_cache(maxsize=None)
def _sc_gather(B: int, num_steps: int):
    info = plsc.get_sparse_core_info()
    nc, ns, L = info.num_cores, info.num_subcores, info.num_lanes
    nw = nc * ns
    b_per_w = B // nw
    mesh = plsc.VectorSubcoreMesh(core_axis_name="c", subcore_axis_name="s")

    @functools.partial(
        pl.kernel,
        mesh=mesh,
        out_type=(
            jax.ShapeDtypeStruct((B,), jnp.float32),
            jax.ShapeDtypeStruct((B,), jnp.float32),
        ),
        scratch_types=[
            pltpu.VMEM((_TABLE_PAD,), jnp.float32),
            pltpu.VMEM((_TABLE_PAD,), jnp.float32),
            pltpu.VMEM((b_per_w,), jnp.int32),
            pltpu.VMEM((b_per_w,), jnp.float32),
            pltpu.VMEM((b_per_w,), jnp.float32),
        ],
        compiler_params=pltpu.CompilerParams(needs_layout_passes=False),
    )
    def gather(t_hbm, sab_hbm, somab_hbm, sa_hbm, sb_hbm,
               sab_v, somab_v, idx_v, sa_v, sb_v):
        wid = lax.axis_index("s") * nc + lax.axis_index("c")
        base = wid * b_per_w
        pltpu.sync_copy(sab_hbm, sab_v)
        pltpu.sync_copy(somab_hbm, somab_v)
        pltpu.sync_copy(t_hbm.at[pl.ds(base, b_per_w)], idx_v)
        for j in range(b_per_w // L):
            idx = idx_v[pl.ds(j * L, L)]
            idx = jnp.minimum(jnp.maximum(idx, 0), num_steps - 1)
            sa_v[pl.ds(j * L, L)] = plsc.load_gather(sab_v, [idx])
            sb_v[pl.ds(j * L, L)] = plsc.load_gather(somab_v, [idx])
        pltpu.sync_copy(sa_v, sa_hbm.at[pl.ds(base, b_per_w)])
        pltpu.sync_copy(sb_v, sb_hbm.at[pl.ds(base, b_per_w)])

    return gather


def _tc_combine_body(sa_ref, sb_ref, x_ref, n_ref, o_ref, no_ref):
    n = n_ref[...]
    o_ref[...] = sa_ref[...] * x_ref[...] + sb_ref[...] * n
    no_ref[...] = n


@functools.lru_cache(maxsize=None)
def _tc_combine(B: int, F: int, R: int):
    data = pl.BlockSpec((R, F), lambda i: (i, 0))
    scale = pl.BlockSpec((R, 1), lambda i: (i, 0))
    shape = jax.ShapeDtypeStruct((B, F), jnp.float32)
    return pl.pallas_call(
        _tc_combine_body,
        grid=(B // R,),
        in_specs=[scale, scale, data, data],
        out_specs=(data, data),
        out_shape=(shape, shape),
    )


def kernel(x_start, noise, t, sqrt_alpha_bars, sqrt_one_minus_alpha_bars):
    B = x_start.shape[0]
    F = x_start.size // B
    num_steps = sqrt_alpha_bars.shape[0]
    pad = _TABLE_PAD - num_steps
    sab = jnp.pad(sqrt_alpha_bars, (0, pad))
    somab = jnp.pad(sqrt_one_minus_alpha_bars, (0, pad))
    sa, sb = _sc_gather(B, num_steps)(t, sab, somab)
    x_t, noise_out = _tc_combine(B, F, 8)(
        sa.reshape(B, 1), sb.reshape(B, 1),
        x_start.reshape(B, F), noise.reshape(B, F))
    return (x_t.reshape(x_start.shape), noise_out.reshape(x_start.shape))


# speedup vs baseline: 1.5633x; 1.1200x over previous
import functools

import jax
import jax.numpy as jnp
from jax import lax
from jax.experimental import pallas as pl
from jax.experimental.pallas import tpu as pltpu
from jax.experimental.pallas import tpu_sc as plsc

_TABLE_PAD = 1024


@functools.lru_cache(maxsize=None)
def _sc_gather(B: int, num_steps: int):
    info = plsc.get_sparse_core_info()
    nc, ns, L = info.num_cores, info.num_subcores, info.num_lanes
    nw = nc * ns
    b_per_w = B // nw
    mesh = plsc.VectorSubcoreMesh(core_axis_name="c", subcore_axis_name="s")

    @functools.partial(
        pl.kernel,
        mesh=mesh,
        out_type=(
            jax.ShapeDtypeStruct((B,), jnp.float32),
            jax.ShapeDtypeStruct((B,), jnp.float32),
        ),
        scratch_types=[
            pltpu.VMEM((_TABLE_PAD,), jnp.float32),
            pltpu.VMEM((_TABLE_PAD,), jnp.float32),
            pltpu.VMEM((b_per_w,), jnp.int32),
            pltpu.VMEM((b_per_w,), jnp.float32),
            pltpu.VMEM((b_per_w,), jnp.float32),
        ],
        compiler_params=pltpu.CompilerParams(needs_layout_passes=False),
    )
    def gather(t_hbm, sab_hbm, somab_hbm, sa_hbm, sb_hbm,
               sab_v, somab_v, idx_v, sa_v, sb_v):
        wid = lax.axis_index("s") * nc + lax.axis_index("c")
        base = wid * b_per_w
        pltpu.sync_copy(sab_hbm, sab_v)
        pltpu.sync_copy(somab_hbm, somab_v)
        pltpu.sync_copy(t_hbm.at[pl.ds(base, b_per_w)], idx_v)
        for j in range(b_per_w // L):
            idx = idx_v[pl.ds(j * L, L)]
            idx = jnp.minimum(jnp.maximum(idx, 0), num_steps - 1)
            sa_v[pl.ds(j * L, L)] = plsc.load_gather(sab_v, [idx])
            sb_v[pl.ds(j * L, L)] = plsc.load_gather(somab_v, [idx])
        pltpu.sync_copy(sa_v, sa_hbm.at[pl.ds(base, b_per_w)])
        pltpu.sync_copy(sb_v, sb_hbm.at[pl.ds(base, b_per_w)])

    return gather


def _tc_combine_body(sa_ref, sb_ref, x_ref, n_ref, o_ref):
    o_ref[...] = sa_ref[...] * x_ref[...] + sb_ref[...] * n_ref[...]


@functools.lru_cache(maxsize=None)
def _tc_combine(B: int, F: int, R: int):
    data = pl.BlockSpec((R, F), lambda i: (i, 0))
    scale = pl.BlockSpec((R, 1), lambda i: (i, 0))
    shape = jax.ShapeDtypeStruct((B, F), jnp.float32)
    return pl.pallas_call(
        _tc_combine_body,
        grid=(B // R,),
        in_specs=[scale, scale, data, data],
        out_specs=data,
        out_shape=shape,
    )


def kernel(x_start, noise, t, sqrt_alpha_bars, sqrt_one_minus_alpha_bars):
    B = x_start.shape[0]
    F = x_start.size // B
    num_steps = sqrt_alpha_bars.shape[0]
    t_clip = jnp.clip(t, 0, num_steps - 1)
    sa = jnp.take(sqrt_alpha_bars, t_clip)
    sb = jnp.take(sqrt_one_minus_alpha_bars, t_clip)
    x_t = _tc_combine(B, F, 8)(
        sa.reshape(B, 1), sb.reshape(B, 1),
        x_start.reshape(B, F), noise.reshape(B, F))
    return (x_t.reshape(x_start.shape), noise)
